# baseline (speedup 1.0000x reference)
import functools

import jax
import jax.numpy as jnp
from jax import lax
from jax.experimental import pallas as pl
from jax.experimental.pallas import tpu as pltpu
from jax.experimental.pallas import tpu_sc as plsc

ROWS = 128
N = 32768
LANES = 16
NCHUNK = N // LANES
NUM_CORES = 2
NUM_SUBCORES = 16
NUM_WORKERS = NUM_CORES * NUM_SUBCORES
ROWS_PER_W = ROWS // NUM_WORKERS

_mesh = plsc.VectorSubcoreMesh(
    core_axis_name="c", subcore_axis_name="s",
    num_cores=NUM_CORES, num_subcores=NUM_SUBCORES)


def _sparsemax_body(x_hbm, out_hbm, row_a, row_b, cand_v,
                    si_a, si_b, so_a, so_b):
    wid = lax.axis_index("s") * NUM_CORES + lax.axis_index("c")
    base_r = wid * ROWS_PER_W
    bufs = [row_a, row_b]
    sin = [si_a, si_b]
    sout = [so_a, so_b]

    cins = {0: pltpu.async_copy(x_hbm.at[base_r], row_a, si_a)}
    couts = {}
    for i in range(ROWS_PER_W):
        row_v = bufs[i % 2]
        cins[i].wait()
        if i + 1 < ROWS_PER_W:
            if i - 1 >= 0:
                couts[i - 1].wait()
            cins[i + 1] = pltpu.async_copy(
                x_hbm.at[base_r + i + 1], bufs[(i + 1) % 2],
                sin[(i + 1) % 2])

        @plsc.parallel_loop(0, N, step=LANES, unroll=8,
                            carry=jnp.full((LANES,), -jnp.inf, jnp.float32))
        def acc(i2, a):
            return jnp.maximum(
                a, row_v[pl.ds(pl.multiple_of(i2, LANES), LANES)])
        m = jnp.max(acc)
        thr = m - 1.0

        @plsc.parallel_loop(0, N, step=LANES, unroll=8,
                            carry=(jnp.zeros((LANES,), jnp.int32),
                                   jnp.zeros((LANES,), jnp.float32)))
        def off_sv(i2, state):
            v = row_v[pl.ds(pl.multiple_of(i2, LANES), LANES)]
            msk = v > thr

            def have(st):
                ov, sv = st
                pos = plsc.cumsum(msk.astype(jnp.int32)) - 1 + ov
                plsc.store_scatter(cand_v, [pos], v, mask=msk)
                return (ov + plsc.all_reduce_population_count(msk),
                        sv + jnp.where(msk, v, 0.0))

            return lax.cond(jnp.any(msk), have, lambda st: st, state)

        off_vec, sv0 = off_sv
        k_cand = jnp.max(off_vec)
        s0 = jnp.sum(sv0)
        pad_idx = off_vec + lax.iota(jnp.int32, LANES)
        plsc.store_scatter(cand_v, [pad_idx],
                           jnp.full((LANES,), thr, jnp.float32))
        nch = lax.shift_right_logical(k_cand + (LANES - 1), 4)

        def fstate(t):
            def body(j, sc):
                s, c = sc
                v = cand_v[pl.ds(pl.multiple_of(j * LANES, LANES), LANES)]
                msk = v > t
                return (s + jnp.where(msk, v, 0.0), c + msk.astype(jnp.int32))
            sv, cv = lax.fori_loop(
                0, nch, body,
                (jnp.zeros((LANES,), jnp.float32),
                 jnp.zeros((LANES,), jnp.int32)))
            return jnp.sum(sv), jnp.sum(cv)

        def bis_cond(st):
            it, lo, hi, s_lo, c_lo, c_hi = st
            return jnp.logical_and(it < 30, c_lo != c_hi)

        def bis_body(st):
            it, lo, hi, s_lo, c_lo, c_hi = st
            mid = 0.5 * (lo + hi)
            s_m, c_m = fstate(mid)
            gt = s_m - mid * c_m.astype(jnp.float32) > 1.0
            return (it + 1,
                    jnp.where(gt, mid, lo), jnp.where(gt, hi, mid),
                    jnp.where(gt, s_m, s_lo), jnp.where(gt, c_m, c_lo),
                    jnp.where(gt, c_hi, c_m))

        _, lo, hi, s_lo, c_lo, c_hi = lax.while_loop(
            bis_cond, bis_body,
            (jnp.int32(0), thr, m, s0, k_cand, jnp.int32(0)))

        s_v = jnp.full((LANES,), s_lo - 1.0, jnp.float32)
        c_v = jnp.full((LANES,), c_lo, jnp.int32).astype(jnp.float32)
        tau_v = s_v / c_v

        @plsc.parallel_loop(0, N, step=LANES, unroll=8)
        def _(i2):
            jslice = pl.ds(pl.multiple_of(i2, LANES), LANES)
            row_v[jslice] = jnp.maximum(row_v[jslice] - tau_v, 0.0)

        couts[i] = pltpu.async_copy(
            row_v, out_hbm.at[base_r + i], sout[i % 2])

    couts[ROWS_PER_W - 2].wait()
    couts[ROWS_PER_W - 1].wait()


_sparsemax = functools.partial(
    pl.kernel,
    out_type=jax.ShapeDtypeStruct((ROWS, N), jnp.float32),
    mesh=_mesh,
    scratch_types=[
        pltpu.VMEM((N,), jnp.float32),
        pltpu.VMEM((N,), jnp.float32),
        pltpu.VMEM((N + LANES,), jnp.float32),
        pltpu.SemaphoreType.DMA,
        pltpu.SemaphoreType.DMA,
        pltpu.SemaphoreType.DMA,
        pltpu.SemaphoreType.DMA,
    ],
    compiler_params=pltpu.CompilerParams(needs_layout_passes=False),
)(_sparsemax_body)


@jax.jit
def kernel(x):
    return _sparsemax(x)
